# Initial kernel scaffold; baseline (speedup 1.0000x reference)
import jax
import jax.numpy as jnp
from jax.experimental import pallas as pl


def kernel(inputs, adj_indices, adj_values, W, b):
    raise NotImplementedError("write your pallas kernel here")


# speedup vs baseline: 3.0469x; 3.0469x over previous
import functools

import jax
import jax.numpy as jnp
from jax import lax
from jax.experimental import pallas as pl
from jax.experimental.pallas import tpu as pltpu
from jax.experimental.pallas import tpu_sc as plsc

NC = 2
NS = 16
L = 16
C = 128


def _mm_tanh_kernel(x_ref, w_ref, o_ref):
    o_ref[...] = jnp.tanh(
        jnp.dot(x_ref[...], w_ref[...], preferred_element_type=jnp.float32))


def _finish_kernel(p_ref, b_ref, o_ref):
    o_ref[...] = jnp.tanh(p_ref[0] + p_ref[1]) + b_ref[...]


def _sc_spmm(n_rows, chunks_per_tile, h, src, dst, val, zeros):
    d = h.shape[1]
    rows_per_tile = n_rows // NS
    mesh = plsc.VectorSubcoreMesh(core_axis_name="c", subcore_axis_name="s")

    @functools.partial(
        pl.kernel,
        out_type=jax.ShapeDtypeStruct((NC, n_rows, d), jnp.float32),
        mesh=mesh,
        scratch_types=[
            pltpu.VMEM((C,), jnp.int32),
            pltpu.VMEM((C,), jnp.int32),
            pltpu.VMEM((C, L), jnp.float32),
            pltpu.VMEM((C, d), jnp.float32),
            pltpu.VMEM_SHARED((n_rows, d), jnp.float32),
            pltpu.SemaphoreType.DMA,
        ],
    )
    def spmm(h_hbm, src_hbm, dst_hbm, val_hbm, z_hbm, out_hbm,
             src_v, dst_v, val_v, rows_v, agg_sh, sem):
        cid = lax.axis_index("c")
        sid = lax.axis_index("s")
        wid = cid * NS + sid

        row0 = sid * rows_per_tile
        pltpu.sync_copy(z_hbm, agg_sh.at[pl.ds(row0, rows_per_tile)])
        plsc.subcore_barrier()

        edge0 = wid * (chunks_per_tile * C)

        def chunk_body(i, carry):
            base = edge0 + i * C
            pltpu.sync_copy(src_hbm.at[pl.ds(base, C)], src_v)
            pltpu.sync_copy(dst_hbm.at[pl.ds(base, C)], dst_v)
            pltpu.sync_copy(val_hbm.at[pl.ds(base, C)], val_v)
            pltpu.async_copy(h_hbm.at[src_v], rows_v, sem).wait()

            def scale_body(e, carry2):
                vsplat = val_v[e, :]
                for j in range(d // L):
                    sl = pl.ds(j * L, L)
                    rows_v[e, sl] = rows_v[e, sl] * vsplat
                return carry2

            lax.fori_loop(0, C, scale_body, 0)
            pltpu.sync_copy(rows_v, agg_sh.at[dst_v], add=True)
            return carry

        lax.fori_loop(0, chunks_per_tile, chunk_body, 0)
        plsc.subcore_barrier()
        pltpu.sync_copy(agg_sh.at[pl.ds(row0, rows_per_tile)],
                        out_hbm.at[cid, pl.ds(row0, rows_per_tile)])

    return spmm(h, src, dst, val, zeros)


def kernel(inputs, adj_indices, adj_values, W, b):
    n, d = inputs.shape
    e = adj_values.shape[0]

    blk = 1000
    h = pl.pallas_call(
        _mm_tanh_kernel,
        grid=(n // blk,),
        in_specs=[
            pl.BlockSpec((blk, d), lambda i: (i, 0)),
            pl.BlockSpec((d, d), lambda i: (0, 0)),
        ],
        out_specs=pl.BlockSpec((blk, d), lambda i: (i, 0)),
        out_shape=jax.ShapeDtypeStruct((n, d), jnp.float32),
    )(inputs, W)

    tile_quota = NC * NS * C
    ep = ((e + tile_quota - 1) // tile_quota) * tile_quota
    pad = ep - e
    src = jnp.concatenate(
        [adj_indices[1], jnp.zeros((pad,), jnp.int32)])
    dst = jnp.concatenate(
        [adj_indices[0], jnp.zeros((pad,), jnp.int32)])
    val = jnp.broadcast_to(
        jnp.concatenate([adj_values, jnp.zeros((pad,), jnp.float32)])[:, None],
        (ep, L))
    n_pad = ((n + NS * 8 - 1) // (NS * 8)) * (NS * 8)
    zeros = jnp.zeros((n_pad // NS, d), jnp.float32)
    partials = _sc_spmm(n_pad, ep // (NC * NS * C), h, src, dst, val, zeros)

    out = pl.pallas_call(
        _finish_kernel,
        grid=(n // blk,),
        in_specs=[
            pl.BlockSpec((NC, blk, d), lambda i: (0, i, 0)),
            pl.BlockSpec((d,), lambda i: (0,)),
        ],
        out_specs=pl.BlockSpec((blk, d), lambda i: (i, 0)),
        out_shape=jax.ShapeDtypeStruct((n, d), jnp.float32),
    )(partials, b)
    return out
